# baseline (speedup 1.0000x reference)
import functools

import jax
import jax.numpy as jnp
from jax import lax
from jax.experimental import pallas as pl
from jax.experimental.pallas import tpu as pltpu
from jax.experimental.pallas import tpu_sc as plsc

_VOCAB = 1000000
_D = 32
_BATCH = 16384
_HIST = 50
_NC = 2
_NS = 16
_NW = _NC * _NS
_BBLK = _BATCH // 128
_CPW = _BBLK // _NW
_NU = _HIST * _CPW

_mesh = plsc.VectorSubcoreMesh(core_axis_name="c", subcore_axis_name="s")


@functools.partial(
    pl.kernel,
    out_type=jax.ShapeDtypeStruct((_HIST, _D // 8, 128, 8, 128), jnp.float32),
    mesh=_mesh,
    scratch_types=[
        pltpu.VMEM((_HIST, _CPW * 128), jnp.int32),
        pltpu.VMEM((8, 128, _D), jnp.float32),
        pltpu.VMEM((8, _D // 8, 8, 128), jnp.float32),
        [pltpu.SemaphoreType.DMA for _ in range(8)],
        [pltpu.SemaphoreType.DMA for _ in range(8)],
    ],
    compiler_params=pltpu.CompilerParams(use_tc_tiling_on_sc=False, needs_layout_passes=False),
)
def _gather(ids_hbm, table_hbm, out_hbm, idx_v, rows_v, tblk_v,
            gsems, osems):
    wid = lax.axis_index("s") * _NC + lax.axis_index("c")
    col0 = wid * (_CPW * 128)
    pltpu.sync_copy(ids_hbm.at[:, pl.ds(col0, _CPW * 128)], idx_v)

    lanes = lax.iota(jnp.int32, 16)
    rvecs = [lanes + (bg * 16) for bg in range(8)]

    def fire_gather(u, half, sem):
        t = u // _CPW
        c = u % _CPW
        pltpu.async_copy(table_hbm.at[idx_v.at[t, pl.ds(c * 128, 128)]],
                         rows_v.at[half], sem)

    def drain_gather(u, half, sem):
        t = u // _CPW
        c = u % _CPW
        pltpu.make_async_copy(table_hbm.at[idx_v.at[t, pl.ds(c * 128, 128)]],
                              rows_v.at[half], sem).wait()

    def transpose(half):
        @pl.loop(0, _D // 16)
        def _j(jq):
            j0 = 16 * jq
            cvecs = [jnp.zeros((16,), jnp.int32) + (j0 + q) for q in range(16)]
            for bg in range(8):
                vals = [plsc.load_gather(rows_v.at[half], [rvecs[bg], cvecs[q]])
                        for q in range(16)]
                for q in range(16):
                    tblk_v.at[half, 2 * jq + q // 8, q % 8][pl.ds(bg * 16, 16)] = vals[q]

    def fire_out(u, half, sem):
        t = u // _CPW
        c = u % _CPW
        pltpu.async_copy(tblk_v.at[half],
                         out_hbm.at[t, :, wid * _CPW + c], sem)

    def drain_out(u, half, sem):
        t = u // _CPW
        c = u % _CPW
        pltpu.make_async_copy(tblk_v.at[half],
                              out_hbm.at[t, :, wid * _CPW + c], sem).wait()

    for q in range(8):
        fire_gather(q, q, gsems[q])

    @pl.loop(0, _NU // 8)
    def _quad(g):
        for q in range(8):
            u = 8 * g + q
            drain_gather(u, q, gsems[q])

            @pl.when(g > 0)
            def _():
                drain_out(u - 8, q, osems[q])

            transpose(q)

            @pl.when(g + 1 < _NU // 8)
            def _():
                fire_gather(u + 8, q, gsems[q])

            fire_out(u, q, osems[q])

    for q in range(8):
        drain_out(_NU - 8 + q, q, osems[q])


def kernel(ids, length, table):
    del length
    out5 = _gather(ids.T, table)
    return jnp.transpose(out5, (2, 4, 0, 1, 3)).reshape(_BATCH, _HIST, _D)


# speedup vs baseline: 1.0051x; 1.0051x over previous
import functools

import jax
import jax.numpy as jnp
from jax import lax
from jax.experimental import pallas as pl
from jax.experimental.pallas import tpu as pltpu
from jax.experimental.pallas import tpu_sc as plsc

_VOCAB = 1000000
_D = 32
_BATCH = 16384
_HIST = 50
_NC = 2
_NS = 16
_NW = _NC * _NS
_BBLK = _BATCH // 128
_CPW = _BBLK // _NW
_NU = _HIST * _CPW

_mesh = plsc.VectorSubcoreMesh(core_axis_name="c", subcore_axis_name="s")


@functools.partial(
    pl.kernel,
    out_type=jax.ShapeDtypeStruct((_HIST, _D // 8, 128, 8, 128), jnp.float32),
    mesh=_mesh,
    scratch_types=[
        pltpu.VMEM((_HIST, _CPW * 128), jnp.int32),
        pltpu.VMEM((4, 128, _D), jnp.float32),
        pltpu.VMEM((4, _D // 8, 8, 128), jnp.float32),
        [pltpu.SemaphoreType.DMA for _ in range(4)],
        [pltpu.SemaphoreType.DMA for _ in range(4)],
    ],
    compiler_params=pltpu.CompilerParams(use_tc_tiling_on_sc=False, needs_layout_passes=False),
)
def _gather(ids_hbm, table_hbm, out_hbm, idx_v, rows_v, tblk_v,
            gsems, osems):
    wid = lax.axis_index("s") * _NC + lax.axis_index("c")
    col0 = wid * (_CPW * 128)
    pltpu.sync_copy(ids_hbm.at[:, pl.ds(col0, _CPW * 128)], idx_v)

    lanes = lax.iota(jnp.int32, 16)
    rvecs = [lanes + (bg * 16) for bg in range(8)]

    def fire_gather(u, half, sem):
        t = u // _CPW
        c = u % _CPW
        pltpu.async_copy(table_hbm.at[idx_v.at[t, pl.ds(c * 128, 128)]],
                         rows_v.at[half], sem)

    def drain_gather(u, half, sem):
        t = u // _CPW
        c = u % _CPW
        pltpu.make_async_copy(table_hbm.at[idx_v.at[t, pl.ds(c * 128, 128)]],
                              rows_v.at[half], sem).wait()

    def transpose(half):
        @pl.loop(0, _D // 16)
        def _j(jq):
            j0 = 16 * jq
            cvecs = [jnp.zeros((16,), jnp.int32) + (j0 + q) for q in range(16)]
            for bg in range(8):
                vals = [plsc.load_gather(rows_v.at[half], [rvecs[bg], cvecs[q]])
                        for q in range(16)]
                for q in range(16):
                    tblk_v.at[half, 2 * jq + q // 8, q % 8][pl.ds(bg * 16, 16)] = vals[q]

    def fire_out(u, half, sem):
        t = u // _CPW
        c = u % _CPW
        pltpu.async_copy(tblk_v.at[half],
                         out_hbm.at[t, :, wid * _CPW + c], sem)

    def drain_out(u, half, sem):
        t = u // _CPW
        c = u % _CPW
        pltpu.make_async_copy(tblk_v.at[half],
                              out_hbm.at[t, :, wid * _CPW + c], sem).wait()

    for q in range(4):
        fire_gather(q, q, gsems[q])

    @pl.loop(0, _NU // 4)
    def _quad(g):
        for q in range(4):
            u = 4 * g + q
            drain_gather(u, q, gsems[q])

            @pl.when(g > 0)
            def _():
                drain_out(u - 4, q, osems[q])

            transpose(q)

            @pl.when(g + 1 < _NU // 4)
            def _():
                fire_gather(u + 4, q, gsems[q])

            fire_out(u, q, osems[q])

    for q in range(4):
        drain_out(_NU - 4 + q, q, osems[q])


def kernel(ids, length, table):
    del length
    out5 = _gather(ids.T, table)
    return jnp.transpose(out5, (2, 4, 0, 1, 3)).reshape(_BATCH, _HIST, _D)
